# baseline (speedup 1.0000x reference)
import jax
import jax.numpy as jnp
from jax import lax
from jax.experimental import pallas as pl
from jax.experimental.pallas import tpu as pltpu
from jax.experimental.pallas import tpu_sc as plsc

D_MODEL = 64
SCALE = 8.0

NUM_CORES = 2
NUM_SUBCORES = 16
LANES = 16
NUM_WORKERS = NUM_CORES * NUM_SUBCORES

BW = 128
NBUF = 4


def _emb_body(xt_hbm, table_hbm, out_hbm, idx_v, rows, trans, gsems, wsems):
    seq = xt_hbm.shape[0]
    wid = lax.axis_index("s") * NUM_CORES + lax.axis_index("c")
    b0 = wid * BW
    pltpu.sync_copy(xt_hbm.at[:, pl.ds(b0, BW)], idx_v)

    def start_gather(i, b):
        pltpu.async_copy(table_hbm.at[idx_v.at[i]], rows[b], gsems[b])

    for b in range(NBUF):
        start_gather(b, b)

    def process(i, b):
        pltpu.make_async_copy(table_hbm.at[idx_v.at[i]], rows[b],
                              gsems[b]).wait()
        @pl.when(i >= NBUF)
        def _():
            pltpu.make_async_copy(
                trans[b], out_hbm.at[i, :, pl.ds(b0, BW)], wsems[b]).wait()

        def group(g, carry):
            idxvec = idx_v[i, pl.ds(g * LANES, LANES)]
            svec = jnp.where(idxvec == 0, 0.0, SCALE).astype(jnp.float32)
            rowvec = g * LANES + lax.iota(jnp.int32, LANES)
            for d in range(D_MODEL):
                colvec = jnp.full((LANES,), d, jnp.int32)
                val = plsc.load_gather(rows[b], [rowvec, colvec]) * svec
                trans[b][d, pl.ds(g * LANES, LANES)] = val
            return carry

        lax.fori_loop(0, BW // LANES, group, 0)
        pltpu.async_copy(trans[b], out_hbm.at[i, :, pl.ds(b0, BW)], wsems[b])

        @pl.when(i + NBUF < seq)
        def _():
            start_gather(i + NBUF, b)

    def outer(g, carry):
        for b in range(NBUF):
            process(g * NBUF + b, b)
        return carry

    lax.fori_loop(0, seq // NBUF, outer, 0)
    for b in range(NBUF):
        i = seq - NBUF + b
        pltpu.make_async_copy(
            trans[b], out_hbm.at[i, :, pl.ds(b0, BW)], wsems[b]).wait()


def kernel(x, table):
    bsz, seq = x.shape
    xt = x.T
    k = pl.kernel(
        _emb_body,
        out_type=jax.ShapeDtypeStruct((seq, D_MODEL, bsz), jnp.float32),
        mesh=plsc.VectorSubcoreMesh(
            core_axis_name="c", subcore_axis_name="s"),
        scratch_types=[
            pltpu.VMEM((seq, BW), jnp.int32),
            [pltpu.VMEM((BW, D_MODEL), jnp.float32) for _ in range(NBUF)],
            [pltpu.VMEM((D_MODEL, BW), jnp.float32) for _ in range(NBUF)],
            [pltpu.SemaphoreType.DMA for _ in range(NBUF)],
            [pltpu.SemaphoreType.DMA for _ in range(NBUF)],
        ],
        compiler_params=pltpu.CompilerParams(
            use_tc_tiling_on_sc=False, needs_layout_passes=False),
    )
    out_t = k(xt, table)
    return jnp.transpose(out_t, (2, 0, 1))


# speedup vs baseline: 1.5098x; 1.5098x over previous
import jax
import jax.numpy as jnp
from jax import lax
from jax.experimental import pallas as pl
from jax.experimental.pallas import tpu as pltpu
from jax.experimental.pallas import tpu_sc as plsc

D_MODEL = 64
SCALE = 8.0

NUM_CORES = 2
NUM_SUBCORES = 16
LANES = 16
NUM_WORKERS = NUM_CORES * NUM_SUBCORES

BW = 128
NBUF = 4


def _emb_body(xt_hbm, table_hbm, out_hbm, idx_v, rows, trans, gsems, wsems):
    seq = xt_hbm.shape[0]
    wid = lax.axis_index("s") * NUM_CORES + lax.axis_index("c")
    b0 = wid * BW
    pltpu.sync_copy(xt_hbm.at[:, pl.ds(b0, BW)], idx_v)

    def start_gather(i, b):
        pltpu.async_copy(table_hbm.at[idx_v.at[i]], rows[b], gsems[b])

    for b in range(NBUF):
        start_gather(b, b)

    def process(i, b):
        pltpu.make_async_copy(table_hbm.at[idx_v.at[i]], rows[b],
                              gsems[b]).wait()
        @pl.when(i >= NBUF)
        def _():
            pltpu.make_async_copy(
                trans[b], out_hbm.at[i, :, pl.ds(b0, BW)], wsems[b]).wait()

        def group(g, carry):
            idxvec = idx_v[i, pl.ds(g * LANES, LANES)]
            svec = jnp.where(idxvec == 0, 0.0, SCALE).astype(jnp.float32)
            lanes = lax.iota(jnp.int32, LANES)
            rowvec = g * LANES + lanes
            for k in range(D_MODEL):
                colvec = (lanes + k) & (D_MODEL - 1)
                val = plsc.load_gather(rows[b], [rowvec, colvec]) * svec
                plsc.store_scatter(trans[b], [colvec, rowvec], val)
            return carry

        lax.fori_loop(0, BW // LANES, group, 0)
        pltpu.async_copy(trans[b], out_hbm.at[i, :, pl.ds(b0, BW)], wsems[b])

        @pl.when(i + NBUF < seq)
        def _():
            start_gather(i + NBUF, b)

    def outer(g, carry):
        for b in range(NBUF):
            process(g * NBUF + b, b)
        return carry

    lax.fori_loop(0, seq // NBUF, outer, 0)
    for b in range(NBUF):
        i = seq - NBUF + b
        pltpu.make_async_copy(
            trans[b], out_hbm.at[i, :, pl.ds(b0, BW)], wsems[b]).wait()


def kernel(x, table):
    bsz, seq = x.shape
    xt = x.T
    k = pl.kernel(
        _emb_body,
        out_type=jax.ShapeDtypeStruct((seq, D_MODEL, bsz), jnp.float32),
        mesh=plsc.VectorSubcoreMesh(
            core_axis_name="c", subcore_axis_name="s"),
        scratch_types=[
            pltpu.VMEM((seq, BW), jnp.int32),
            [pltpu.VMEM((BW, D_MODEL), jnp.float32) for _ in range(NBUF)],
            [pltpu.VMEM((D_MODEL, BW), jnp.float32) for _ in range(NBUF)],
            [pltpu.SemaphoreType.DMA for _ in range(NBUF)],
            [pltpu.SemaphoreType.DMA for _ in range(NBUF)],
        ],
        compiler_params=pltpu.CompilerParams(
            use_tc_tiling_on_sc=False, needs_layout_passes=False),
    )
    out_t = k(xt, table)
    return jnp.transpose(out_t, (2, 0, 1))
